# baseline (speedup 1.0000x reference)
import functools

import jax
import jax.numpy as jnp
from jax import lax
from jax.experimental import pallas as pl
from jax.experimental.pallas import tpu as pltpu
from jax.experimental.pallas import tpu_sc as plsc


@functools.cache
def _make_gather4(B, F):
    info = plsc.get_sparse_core_info()
    nw = info.num_cores * info.num_subcores
    assert B % (8 * nw) == 0
    bpw = B // nw
    mesh = plsc.VectorSubcoreMesh(core_axis_name="c", subcore_axis_name="s")
    f32 = jnp.float32

    @functools.partial(
        pl.kernel,
        mesh=mesh,
        compiler_params=pltpu.CompilerParams(use_tc_tiling_on_sc=False),
        out_type=[jax.ShapeDtypeStruct((B, F), f32)] * 4,
        scratch_types=[
            pltpu.VMEM((bpw,), jnp.int32),
            pltpu.VMEM((bpw,), jnp.int32),
            pltpu.VMEM((bpw, F), f32),
            pltpu.VMEM((bpw, F), f32),
            pltpu.VMEM((bpw, F), f32),
            pltpu.VMEM((bpw, F), f32),
            pltpu.SemaphoreType.DMA,
            pltpu.SemaphoreType.DMA,
            pltpu.SemaphoreType.DMA,
            pltpu.SemaphoreType.DMA,
        ],
    )
    def gather4(ug_h, ig_h, um_h, im_h, uidx_h, iidx_h,
                o_ug, o_ig, o_um, o_im,
                uidx_v, iidx_v, r_ug, r_ig, r_um, r_im, s0, s1, s2, s3):
        wid = lax.axis_index("s") * info.num_cores + lax.axis_index("c")
        base = wid * bpw
        pltpu.sync_copy(uidx_h.at[pl.ds(base, bpw)], uidx_v)
        pltpu.sync_copy(iidx_h.at[pl.ds(base, bpw)], iidx_v)
        c0 = pltpu.async_copy(ug_h.at[uidx_v], r_ug, s0)
        c1 = pltpu.async_copy(ig_h.at[iidx_v], r_ig, s1)
        c2 = pltpu.async_copy(um_h.at[uidx_v], r_um, s2)
        c3 = pltpu.async_copy(im_h.at[iidx_v], r_im, s3)
        c0.wait()
        pltpu.sync_copy(r_ug, o_ug.at[pl.ds(base, bpw)])
        c1.wait()
        pltpu.sync_copy(r_ig, o_ig.at[pl.ds(base, bpw)])
        c2.wait()
        pltpu.sync_copy(r_um, o_um.at[pl.ds(base, bpw)])
        c3.wait()
        pltpu.sync_copy(r_im, o_im.at[pl.ds(base, bpw)])

    return gather4


_TB = 512


def _gelu(x):
    return 0.5 * x * (1.0 + lax.erf(x * 0.7071067811865476))


def _mlp_body(mu_ref, mi_ref, gu_ref, gi_ref,
              w1u_ref, w1i_ref, b1_ref, w2_ref, b2_ref, w3_ref, b3_ref,
              w4_ref, b4_ref, wfg_ref, wfm_ref, bf_ref, out_ref):
    f32 = jnp.float32
    h = _gelu(jnp.dot(mu_ref[...], w1u_ref[...], preferred_element_type=f32)
              + jnp.dot(mi_ref[...], w1i_ref[...], preferred_element_type=f32)
              + b1_ref[...])
    h = _gelu(jnp.dot(h, w2_ref[...], preferred_element_type=f32) + b2_ref[...])
    h = _gelu(jnp.dot(h, w3_ref[...], preferred_element_type=f32) + b3_ref[...])
    m = jnp.dot(h, w4_ref[...], preferred_element_type=f32) + b4_ref[...]
    g = gu_ref[...] * gi_ref[...]
    out_ref[...] = (jnp.dot(g, wfg_ref[...], preferred_element_type=f32)
                    + jnp.dot(m, wfm_ref[...], preferred_element_type=f32)
                    + bf_ref[...])


def _fused_tail(mu, mi, gu, gi, w1u, w1i, b1, w2, b2, w3, b3, w4, b4,
                wfg, wfm, bf):
    B = mu.shape[0]
    tb = _TB
    grid = (B // tb,)

    def tile():
        return pl.BlockSpec((tb, 16), lambda i: (i, 0))

    def full(shape):
        return pl.BlockSpec(shape, lambda i: (0,) * len(shape))

    return pl.pallas_call(
        _mlp_body,
        grid=grid,
        in_specs=[
            tile(), tile(), tile(), tile(),
            full(w1u.shape), full(w1i.shape), full(b1.shape),
            full(w2.shape), full(b2.shape),
            full(w3.shape), full(b3.shape),
            full(w4.shape), full(b4.shape),
            full(wfg.shape), full(wfm.shape), full(bf.shape),
        ],
        out_specs=pl.BlockSpec((tb, 1), lambda i: (i, 0)),
        out_shape=jax.ShapeDtypeStruct((B, 1), jnp.float32),
    )(mu, mi, gu, gi, w1u, w1i, b1, w2, b2, w3, b3, w4, b4, wfg, wfm, bf)


def kernel(data, user_gmf_w, item_gmf_w, user_mlp_w, item_mlp_w,
           gender_w, authors_w, W1, b1, W2, b2, W3, b3, W4, b4, Wf, bf):
    B = data.shape[0]
    F = user_gmf_w.shape[1]
    users = data[:, 1].astype(jnp.int32)
    items = data[:, 0].astype(jnp.int32)

    gu, gi, mu, mi = _make_gather4(B, F)(
        user_gmf_w, item_gmf_w, user_mlp_w, item_mlp_w, users, items)

    w1t = W1.T
    out2d = _fused_tail(
        mu, mi, gu, gi,
        w1t[:F, :], w1t[F:, :], b1[None, :],
        W2.T, b2[None, :], W3.T, b3[None, :], W4.T, b4[None, :],
        Wf.T[:F, :], Wf.T[F:, :], bf[None, :])
    return out2d[:, 0]
